# Initial kernel scaffold; baseline (speedup 1.0000x reference)
import jax
import jax.numpy as jnp
from jax.experimental import pallas as pl


def kernel(x, edge_index, edge_features, attn, Wz, bz, Wr, br, Wh, bh, lz_w, lz_b, lr_w, lr_b, lh_w, lh_b, l1_w, l1_b, l2_w, l2_b):
    raise NotImplementedError("write your pallas kernel here")


# speedup vs baseline: 213.5986x; 213.5986x over previous
import functools

import jax
import jax.numpy as jnp
from jax import lax
from jax.experimental import pallas as pl
from jax.experimental.pallas import tpu as pltpu
from jax.experimental.pallas import tpu_sc as plsc

N = 10000
NP = 10240
E = 160000
B = 4
F_IN = 128
T = 12
OUT = 32
NB = 200
GRID_N = N // NB

C_CH = 80
C_NCH = 125
A_CH = 128
A_NCH = E // A_CH

_MESH = plsc.VectorSubcoreMesh(core_axis_name="c", subcore_axis_name="s")


def _fill(ref, rows, cols, val):
    def body(r, _):
        for i in range(cols // 16):
            ref[r, pl.ds(i * 16, 16)] = jnp.full((16,), val, jnp.float32)
        return 0
    lax.fori_loop(0, rows, body, 0)


def _fill1d(ref, n, val, dtype=jnp.float32):
    def body(r, _):
        ref[pl.ds(r * 16, 16)] = jnp.full((16,), val, dtype)
        return 0
    lax.fori_loop(0, n // 16, body, 0)


@functools.partial(
    pl.kernel,
    mesh=_MESH,
    out_type=jax.ShapeDtypeStruct((2, NP), jnp.float32),
    scratch_types=[
        pltpu.VMEM((A_CH,), jnp.int32),
        pltpu.VMEM((A_CH,), jnp.float32),
        pltpu.VMEM((640,), jnp.float32),
        pltpu.VMEM_SHARED((NP,), jnp.float32),
    ],
)
def _deg_kernel(dst_hbm, deg_hbm, idx_v, ones_v, zb_v, acc_sh):
    c = lax.axis_index("c")
    s = lax.axis_index("s")
    w = c * 16 + s
    _fill1d(ones_v, A_CH, 1.0)
    _fill1d(zb_v, 640, 0.0)
    pltpu.sync_copy(zb_v, acc_sh.at[pl.ds(s * 640, 640)])
    plsc.subcore_barrier()
    nch = jnp.where(w < 2, 40, 39)

    def body(k, _):
        ci = w + 32 * k
        pltpu.sync_copy(dst_hbm.at[ci], idx_v)
        pltpu.sync_copy(ones_v, acc_sh.at[idx_v], add=True)
        return 0

    lax.fori_loop(0, nch, body, 0)
    plsc.subcore_barrier()
    pltpu.sync_copy(acc_sh.at[pl.ds(s * 640, 640)],
                    deg_hbm.at[c, pl.ds(s * 640, 640)])


@functools.partial(
    pl.kernel,
    mesh=_MESH,
    out_type=jax.ShapeDtypeStruct((4, T, NP, 64), jnp.float32),
    compiler_params=pltpu.CompilerParams(use_tc_tiling_on_sc=False),
    scratch_types=[
        pltpu.VMEM((C_NCH, C_CH), jnp.int32),
        pltpu.VMEM((C_NCH, C_CH), jnp.int32),
        pltpu.VMEM((C_CH,), jnp.int32),
        pltpu.VMEM((C_CH,), jnp.int32),
        pltpu.VMEM((C_CH, 64), jnp.float32),
        pltpu.VMEM((C_CH, 64), jnp.float32),
        pltpu.VMEM((128, 64), jnp.float32),
        pltpu.VMEM_SHARED((NP, 64), jnp.float32),
        pltpu.SemaphoreType.DMA,
        pltpu.SemaphoreType.DMA,
    ],
)
def _agg_kernel(u_hbm, src_hbm, dst_hbm, s_hbm,
                srcv, dstv, adj0, adj1, rows0, rows1, zb, acc_sh,
                sem0, sem1):
    c = lax.axis_index("c")
    s = lax.axis_index("s")
    pltpu.sync_copy(src_hbm.at[s], srcv)
    pltpu.sync_copy(dst_hbm.at[s], dstv)
    _fill(zb, 128, 64, 0.0)
    for i in range(5):
        pltpu.sync_copy(zb, acc_sh.at[pl.ds(s * 640 + i * 128, 128)])
    plsc.subcore_barrier()

    def adj_of(j, off, adj):
        for i in range(C_CH // 16):
            adj[pl.ds(i * 16, 16)] = srcv[j, pl.ds(i * 16, 16)] + off

    def r_body(r, _):
        g = 2 * c + r // T
        off = g * T * N + (r % T) * N

        adj_of(0, off, adj0)
        pltpu.async_copy(u_hbm.at[adj0], rows0, sem0)

        def k_body(k, _):
            c1 = 2 * k + 1
            c2 = 2 * k + 2
            adj_of(c1, off, adj1)
            pltpu.async_copy(u_hbm.at[adj1], rows1, sem1)
            pltpu.make_async_copy(u_hbm.at[adj0], rows0, sem0).wait()
            pltpu.sync_copy(rows0, acc_sh.at[dstv.at[2 * k]], add=True)
            adj_of(c2, off, adj0)
            pltpu.async_copy(u_hbm.at[adj0], rows0, sem0)
            pltpu.make_async_copy(u_hbm.at[adj1], rows1, sem1).wait()
            pltpu.sync_copy(rows1, acc_sh.at[dstv.at[c1]], add=True)
            return 0

        lax.fori_loop(0, (C_NCH - 1) // 2, k_body, 0)
        pltpu.make_async_copy(u_hbm.at[adj0], rows0, sem0).wait()
        pltpu.sync_copy(rows0, acc_sh.at[dstv.at[C_NCH - 1]], add=True)

        plsc.subcore_barrier()
        pltpu.sync_copy(acc_sh.at[pl.ds(s * 640, 640)],
                        s_hbm.at[g, r % T, pl.ds(s * 640, 640)])
        for i in range(5):
            pltpu.sync_copy(zb, acc_sh.at[pl.ds(s * 640 + i * 128, 128)])
        plsc.subcore_barrier()
        return 0

    lax.fori_loop(0, 2 * T, r_body, 0)


def _proj_body(xT_ref, degT_ref, Wz_ref, lzw_ref, Wh_ref, lhw_ref, out_ref):
    Wzp = jnp.dot(Wz_ref[...], lzw_ref[0:OUT, :],
                  preferred_element_type=jnp.float32)
    Whp = jnp.dot(Wh_ref[...], lhw_ref[0:OUT, :],
                  preferred_element_type=jnp.float32)
    d = degT_ref[:, 0:1] + degT_ref[:, 1:2] + 1.0
    dis = lax.rsqrt(d)
    for h in range(2):
        for bl in range(2):
            xb = xT_ref[0, 2 * h + bl]
            yz = jnp.dot(xb, Wzp, preferred_element_type=jnp.float32)
            yh = jnp.dot(xb, Whp, preferred_element_type=jnp.float32)
            out_ref[2 * h, 0, :, bl * OUT:(bl + 1) * OUT] = dis * yz
            out_ref[2 * h + 1, 0, :, bl * OUT:(bl + 1) * OUT] = dis * yh


def _head_body(S_ref, U_ref, degT_ref, attn_ref,
               bz_ref, lzw_ref, lzb_ref, bh_ref, lhw_ref, lhb_ref,
               l1w_ref, l1b_ref, l2w_ref, l2b_ref, out_ref):
    a = attn_ref[0, :]
    m = jnp.max(a)
    e = jnp.exp(a - m)
    p = e / jnp.sum(e)
    cz = jnp.dot(bz_ref[...], lzw_ref[0:OUT, :],
                 preferred_element_type=jnp.float32) + lzb_ref[...]
    ch = jnp.dot(bh_ref[...], lhw_ref[0:OUT, :],
                 preferred_element_type=jnp.float32) + lhb_ref[...]
    d = degT_ref[:, 0:1] + degT_ref[:, 1:2] + 1.0
    dis = lax.rsqrt(d)
    Hacc = [jnp.zeros((NB, OUT), jnp.float32) for _ in range(B)]
    for t in range(T):
        pt = p[t]
        for h in range(2):
            Pz = dis * (S_ref[2 * h, t] + U_ref[2 * h, t])
            Phh = dis * (S_ref[2 * h + 1, t] + U_ref[2 * h + 1, t])
            for bl in range(2):
                b = 2 * h + bl
                preZ = Pz[:, bl * OUT:(bl + 1) * OUT] + cz
                preH = Phh[:, bl * OUT:(bl + 1) * OUT] + ch
                Hn = (1.0 - jax.nn.sigmoid(preZ)) * jnp.tanh(preH)
                Hacc[b] = Hacc[b] + pt * Hn
    for b in range(B):
        h = jnp.tanh(Hacc[b])
        h2 = jnp.tanh(jnp.dot(h, l1w_ref[...],
                              preferred_element_type=jnp.float32)
                      + l1b_ref[...])
        out_ref[b] = jnp.dot(h2, l2w_ref[...],
                             preferred_element_type=jnp.float32) + l2b_ref[...]


def kernel(x, edge_index, edge_features, attn, Wz, bz, Wr, br, Wh, bh,
           lz_w, lz_b, lr_w, lr_b, lh_w, lh_b, l1_w, l1_b, l2_w, l2_b):
    src_r = edge_index[0].reshape(16, C_NCH, C_CH)
    dst_r = edge_index[1].reshape(16, C_NCH, C_CH)
    dst_a = edge_index[1].reshape(A_NCH, A_CH)
    xT = jnp.transpose(x, (3, 0, 1, 2))

    deg2 = _deg_kernel(dst_a)
    degT = jnp.transpose(deg2)

    ubar = pl.pallas_call(
        _proj_body,
        grid=(T, GRID_N),
        in_specs=[
            pl.BlockSpec((1, B, NB, F_IN), lambda t, nb: (t, 0, nb, 0)),
            pl.BlockSpec((NB, 2), lambda t, nb: (nb, 0)),
            pl.BlockSpec((F_IN, OUT), lambda t, nb: (0, 0)),
            pl.BlockSpec((2 * OUT, OUT), lambda t, nb: (0, 0)),
            pl.BlockSpec((F_IN, OUT), lambda t, nb: (0, 0)),
            pl.BlockSpec((2 * OUT, OUT), lambda t, nb: (0, 0)),
        ],
        out_specs=pl.BlockSpec((4, 1, NB, 64), lambda t, nb: (0, t, nb, 0)),
        out_shape=jax.ShapeDtypeStruct((4, T, N, 64), jnp.float32),
    )(xT, degT, Wz, lz_w, Wh, lh_w)

    u_flat = ubar.reshape(4 * T * N, 64)
    s_pad = _agg_kernel(u_flat, src_r, dst_r)

    out = pl.pallas_call(
        _head_body,
        grid=(GRID_N,),
        in_specs=[
            pl.BlockSpec((4, T, NB, 64), lambda nb: (0, 0, nb, 0)),
            pl.BlockSpec((4, T, NB, 64), lambda nb: (0, 0, nb, 0)),
            pl.BlockSpec((NB, 2), lambda nb: (nb, 0)),
            pl.BlockSpec((1, T), lambda nb: (0, 0)),
            pl.BlockSpec((1, OUT), lambda nb: (0, 0)),
            pl.BlockSpec((2 * OUT, OUT), lambda nb: (0, 0)),
            pl.BlockSpec((1, OUT), lambda nb: (0, 0)),
            pl.BlockSpec((1, OUT), lambda nb: (0, 0)),
            pl.BlockSpec((2 * OUT, OUT), lambda nb: (0, 0)),
            pl.BlockSpec((1, OUT), lambda nb: (0, 0)),
            pl.BlockSpec((OUT, 16), lambda nb: (0, 0)),
            pl.BlockSpec((1, 16), lambda nb: (0, 0)),
            pl.BlockSpec((16, 12), lambda nb: (0, 0)),
            pl.BlockSpec((1, 12), lambda nb: (0, 0)),
        ],
        out_specs=pl.BlockSpec((B, NB, 12), lambda nb: (0, nb, 0)),
        out_shape=jax.ShapeDtypeStruct((B, N, 12), jnp.float32),
    )(s_pad, ubar, degT, attn.reshape(1, T),
      bz.reshape(1, OUT), lz_w, lz_b.reshape(1, OUT),
      bh.reshape(1, OUT), lh_w, lh_b.reshape(1, OUT),
      l1_w, l1_b.reshape(1, 16), l2_w, l2_b.reshape(1, 12))
    return out


# speedup vs baseline: 235.3833x; 1.1020x over previous
import functools

import jax
import jax.numpy as jnp
from jax import lax
from jax.experimental import pallas as pl
from jax.experimental.pallas import tpu as pltpu
from jax.experimental.pallas import tpu_sc as plsc

N = 10000
NP = 10240
E = 160000
B = 4
F_IN = 128
T = 12
OUT = 32
NB = 200
GRID_N = N // NB

C_CH = 128
C_NCH = 1250
A_CH = 128
A_NCH = E // A_CH

_MESH = plsc.VectorSubcoreMesh(core_axis_name="c", subcore_axis_name="s")


def _fill(ref, rows, cols, val):
    def body(r, _):
        for i in range(cols // 16):
            ref[r, pl.ds(i * 16, 16)] = jnp.full((16,), val, jnp.float32)
        return 0
    lax.fori_loop(0, rows, body, 0)


def _fill1d(ref, n, val, dtype=jnp.float32):
    def body(r, _):
        ref[pl.ds(r * 16, 16)] = jnp.full((16,), val, dtype)
        return 0
    lax.fori_loop(0, n // 16, body, 0)


@functools.partial(
    pl.kernel,
    mesh=_MESH,
    out_type=jax.ShapeDtypeStruct((2, NP), jnp.float32),
    scratch_types=[
        pltpu.VMEM((A_CH,), jnp.int32),
        pltpu.VMEM((A_CH,), jnp.float32),
        pltpu.VMEM((640,), jnp.float32),
        pltpu.VMEM_SHARED((NP,), jnp.float32),
    ],
)
def _deg_kernel(dst_hbm, deg_hbm, idx_v, ones_v, zb_v, acc_sh):
    c = lax.axis_index("c")
    s = lax.axis_index("s")
    w = c * 16 + s
    _fill1d(ones_v, A_CH, 1.0)
    _fill1d(zb_v, 640, 0.0)
    pltpu.sync_copy(zb_v, acc_sh.at[pl.ds(s * 640, 640)])
    plsc.subcore_barrier()
    nch = jnp.where(w < 2, 40, 39)

    def body(k, _):
        ci = w + 32 * k
        pltpu.sync_copy(dst_hbm.at[ci], idx_v)
        pltpu.sync_copy(ones_v, acc_sh.at[idx_v], add=True)
        return 0

    lax.fori_loop(0, nch, body, 0)
    plsc.subcore_barrier()
    pltpu.sync_copy(acc_sh.at[pl.ds(s * 640, 640)],
                    deg_hbm.at[c, pl.ds(s * 640, 640)])


@functools.partial(
    pl.kernel,
    mesh=_MESH,
    out_type=jax.ShapeDtypeStruct((4, T, NP, 64), jnp.float32),
    compiler_params=pltpu.CompilerParams(use_tc_tiling_on_sc=False),
    scratch_types=[
        pltpu.VMEM((79, 2, C_CH), jnp.int32),
        pltpu.VMEM((C_CH,), jnp.int32),
        pltpu.VMEM((C_CH,), jnp.int32),
        pltpu.VMEM((C_CH, 64), jnp.float32),
        pltpu.VMEM((C_CH, 64), jnp.float32),
        pltpu.VMEM((128, 64), jnp.float32),
        pltpu.VMEM_SHARED((NP, 64), jnp.float32),
        pltpu.SemaphoreType.DMA,
        pltpu.SemaphoreType.DMA,
    ],
)
def _agg_kernel(u_hbm, eidx_hbm, s_hbm,
                ebuf, adj0, adj1, rows0, rows1, zb, acc_sh,
                sem0, sem1):
    c = lax.axis_index("c")
    s = lax.axis_index("s")
    nk = jnp.where(s < 2, 79, 78)

    def pre_body(k, _):
        pltpu.sync_copy(eidx_hbm.at[s + 16 * k], ebuf.at[k])
        return 0

    lax.fori_loop(0, nk, pre_body, 0)
    _fill(zb, 128, 64, 0.0)
    for i in range(5):
        pltpu.sync_copy(zb, acc_sh.at[pl.ds(s * 640 + i * 128, 128)])
    plsc.subcore_barrier()

    def adj_of(j, off, adj):
        for i in range(C_CH // 16):
            adj[pl.ds(i * 16, 16)] = ebuf[j, 0, pl.ds(i * 16, 16)] + off

    def r_body(r, _):
        g = 2 * c + r // T
        off = g * T * N + (r % T) * N

        adj_of(0, off, adj0)
        pltpu.async_copy(u_hbm.at[adj0], rows0, sem0)
        adj_of(1, off, adj1)
        pltpu.async_copy(u_hbm.at[adj1], rows1, sem1)

        def k_body(k, _):
            pltpu.make_async_copy(u_hbm.at[adj0], rows0, sem0).wait()
            pltpu.sync_copy(rows0, acc_sh.at[ebuf.at[2 * k, 1]], add=True)

            @pl.when(k < 38)
            def _():
                adj_of(2 * k + 2, off, adj0)
                pltpu.async_copy(u_hbm.at[adj0], rows0, sem0)

            pltpu.make_async_copy(u_hbm.at[adj1], rows1, sem1).wait()
            pltpu.sync_copy(rows1, acc_sh.at[ebuf.at[2 * k + 1, 1]], add=True)

            @pl.when(k < 38)
            def _():
                adj_of(2 * k + 3, off, adj1)
                pltpu.async_copy(u_hbm.at[adj1], rows1, sem1)

            return 0

        lax.fori_loop(0, 39, k_body, 0)

        @pl.when(s < 2)
        def _():
            adj_of(78, off, adj0)
            pltpu.async_copy(u_hbm.at[adj0], rows0, sem0)
            pltpu.make_async_copy(u_hbm.at[adj0], rows0, sem0).wait()
            pltpu.sync_copy(rows0, acc_sh.at[ebuf.at[78, 1]], add=True)

        plsc.subcore_barrier()
        pltpu.sync_copy(acc_sh.at[pl.ds(s * 640, 640)],
                        s_hbm.at[g, r % T, pl.ds(s * 640, 640)])
        for i in range(5):
            pltpu.sync_copy(zb, acc_sh.at[pl.ds(s * 640 + i * 128, 128)])
        plsc.subcore_barrier()
        return 0

    lax.fori_loop(0, 2 * T, r_body, 0)


def _proj_body(xT_ref, degT_ref, Wz_ref, lzw_ref, Wh_ref, lhw_ref, out_ref):
    Wzp = jnp.dot(Wz_ref[...], lzw_ref[0:OUT, :],
                  preferred_element_type=jnp.float32)
    Whp = jnp.dot(Wh_ref[...], lhw_ref[0:OUT, :],
                  preferred_element_type=jnp.float32)
    d = degT_ref[:, 0:1] + degT_ref[:, 1:2] + 1.0
    dis = lax.rsqrt(d)
    for h in range(2):
        for bl in range(2):
            xb = xT_ref[0, 2 * h + bl]
            yz = jnp.dot(xb, Wzp, preferred_element_type=jnp.float32)
            yh = jnp.dot(xb, Whp, preferred_element_type=jnp.float32)
            out_ref[2 * h, 0, :, bl * OUT:(bl + 1) * OUT] = dis * yz
            out_ref[2 * h + 1, 0, :, bl * OUT:(bl + 1) * OUT] = dis * yh


def _head_body(S_ref, U_ref, degT_ref, attn_ref,
               bz_ref, lzw_ref, lzb_ref, bh_ref, lhw_ref, lhb_ref,
               l1w_ref, l1b_ref, l2w_ref, l2b_ref, out_ref):
    a = attn_ref[0, :]
    m = jnp.max(a)
    e = jnp.exp(a - m)
    p = e / jnp.sum(e)
    cz = jnp.dot(bz_ref[...], lzw_ref[0:OUT, :],
                 preferred_element_type=jnp.float32) + lzb_ref[...]
    ch = jnp.dot(bh_ref[...], lhw_ref[0:OUT, :],
                 preferred_element_type=jnp.float32) + lhb_ref[...]
    d = degT_ref[:, 0:1] + degT_ref[:, 1:2] + 1.0
    dis = lax.rsqrt(d)
    Hacc = [jnp.zeros((NB, OUT), jnp.float32) for _ in range(B)]
    for t in range(T):
        pt = p[t]
        for h in range(2):
            Pz = dis * (S_ref[2 * h, t] + U_ref[2 * h, t])
            Phh = dis * (S_ref[2 * h + 1, t] + U_ref[2 * h + 1, t])
            for bl in range(2):
                b = 2 * h + bl
                preZ = Pz[:, bl * OUT:(bl + 1) * OUT] + cz
                preH = Phh[:, bl * OUT:(bl + 1) * OUT] + ch
                Hn = (1.0 - jax.nn.sigmoid(preZ)) * jnp.tanh(preH)
                Hacc[b] = Hacc[b] + pt * Hn
    for b in range(B):
        h = jnp.tanh(Hacc[b])
        h2 = jnp.tanh(jnp.dot(h, l1w_ref[...],
                              preferred_element_type=jnp.float32)
                      + l1b_ref[...])
        out_ref[b] = jnp.dot(h2, l2w_ref[...],
                             preferred_element_type=jnp.float32) + l2b_ref[...]


def kernel(x, edge_index, edge_features, attn, Wz, bz, Wr, br, Wh, bh,
           lz_w, lz_b, lr_w, lr_b, lh_w, lh_b, l1_w, l1_b, l2_w, l2_b):
    eidx = jnp.stack([edge_index[0].reshape(C_NCH, C_CH),
                      edge_index[1].reshape(C_NCH, C_CH)], axis=1)
    dst_a = edge_index[1].reshape(A_NCH, A_CH)
    xT = jnp.transpose(x, (3, 0, 1, 2))

    deg2 = _deg_kernel(dst_a)
    degT = jnp.transpose(deg2)

    ubar = pl.pallas_call(
        _proj_body,
        grid=(T, GRID_N),
        in_specs=[
            pl.BlockSpec((1, B, NB, F_IN), lambda t, nb: (t, 0, nb, 0)),
            pl.BlockSpec((NB, 2), lambda t, nb: (nb, 0)),
            pl.BlockSpec((F_IN, OUT), lambda t, nb: (0, 0)),
            pl.BlockSpec((2 * OUT, OUT), lambda t, nb: (0, 0)),
            pl.BlockSpec((F_IN, OUT), lambda t, nb: (0, 0)),
            pl.BlockSpec((2 * OUT, OUT), lambda t, nb: (0, 0)),
        ],
        out_specs=pl.BlockSpec((4, 1, NB, 64), lambda t, nb: (0, t, nb, 0)),
        out_shape=jax.ShapeDtypeStruct((4, T, N, 64), jnp.float32),
    )(xT, degT, Wz, lz_w, Wh, lh_w)

    u_flat = ubar.reshape(4 * T * N, 64)
    s_pad = _agg_kernel(u_flat, eidx)

    out = pl.pallas_call(
        _head_body,
        grid=(GRID_N,),
        in_specs=[
            pl.BlockSpec((4, T, NB, 64), lambda nb: (0, 0, nb, 0)),
            pl.BlockSpec((4, T, NB, 64), lambda nb: (0, 0, nb, 0)),
            pl.BlockSpec((NB, 2), lambda nb: (nb, 0)),
            pl.BlockSpec((1, T), lambda nb: (0, 0)),
            pl.BlockSpec((1, OUT), lambda nb: (0, 0)),
            pl.BlockSpec((2 * OUT, OUT), lambda nb: (0, 0)),
            pl.BlockSpec((1, OUT), lambda nb: (0, 0)),
            pl.BlockSpec((1, OUT), lambda nb: (0, 0)),
            pl.BlockSpec((2 * OUT, OUT), lambda nb: (0, 0)),
            pl.BlockSpec((1, OUT), lambda nb: (0, 0)),
            pl.BlockSpec((OUT, 16), lambda nb: (0, 0)),
            pl.BlockSpec((1, 16), lambda nb: (0, 0)),
            pl.BlockSpec((16, 12), lambda nb: (0, 0)),
            pl.BlockSpec((1, 12), lambda nb: (0, 0)),
        ],
        out_specs=pl.BlockSpec((B, NB, 12), lambda nb: (0, nb, 0)),
        out_shape=jax.ShapeDtypeStruct((B, N, 12), jnp.float32),
    )(s_pad, ubar, degT, attn.reshape(1, T),
      bz.reshape(1, OUT), lz_w, lz_b.reshape(1, OUT),
      bh.reshape(1, OUT), lh_w, lh_b.reshape(1, OUT),
      l1_w, l1_b.reshape(1, 16), l2_w, l2_b.reshape(1, 12))
    return out


# speedup vs baseline: 267.6270x; 1.1370x over previous
import functools

import jax
import jax.numpy as jnp
from jax import lax
from jax.experimental import pallas as pl
from jax.experimental.pallas import tpu as pltpu
from jax.experimental.pallas import tpu_sc as plsc

N = 10000
NP = 10240
E = 160000
B = 4
F_IN = 128
T = 12
OUT = 32
NB = 200
GRID_N = N // NB

C_CH = 128
C_NCH = 1250
A_CH = 128
A_NCH = E // A_CH

_MESH = plsc.VectorSubcoreMesh(core_axis_name="c", subcore_axis_name="s")


def _fill(ref, rows, cols, val):
    def body(r, _):
        for i in range(cols // 16):
            ref[r, pl.ds(i * 16, 16)] = jnp.full((16,), val, jnp.float32)
        return 0
    lax.fori_loop(0, rows, body, 0)


def _fill1d(ref, n, val, dtype=jnp.float32):
    def body(r, _):
        ref[pl.ds(r * 16, 16)] = jnp.full((16,), val, dtype)
        return 0
    lax.fori_loop(0, n // 16, body, 0)


@functools.partial(
    pl.kernel,
    mesh=_MESH,
    out_type=jax.ShapeDtypeStruct((2, NP), jnp.float32),
    scratch_types=[
        pltpu.VMEM((A_CH,), jnp.int32),
        pltpu.VMEM((A_CH,), jnp.float32),
        pltpu.VMEM((640,), jnp.float32),
        pltpu.VMEM_SHARED((NP,), jnp.float32),
    ],
)
def _deg_kernel(dst_hbm, deg_hbm, idx_v, ones_v, zb_v, acc_sh):
    c = lax.axis_index("c")
    s = lax.axis_index("s")
    w = c * 16 + s
    _fill1d(ones_v, A_CH, 1.0)
    _fill1d(zb_v, 640, 0.0)
    pltpu.sync_copy(zb_v, acc_sh.at[pl.ds(s * 640, 640)])
    plsc.subcore_barrier()
    nch = jnp.where(w < 2, 40, 39)

    def body(k, _):
        ci = w + 32 * k
        pltpu.sync_copy(dst_hbm.at[ci], idx_v)
        pltpu.sync_copy(ones_v, acc_sh.at[idx_v], add=True)
        return 0

    lax.fori_loop(0, nch, body, 0)
    plsc.subcore_barrier()
    pltpu.sync_copy(acc_sh.at[pl.ds(s * 640, 640)],
                    deg_hbm.at[c, pl.ds(s * 640, 640)])


@functools.partial(
    pl.kernel,
    mesh=_MESH,
    out_type=jax.ShapeDtypeStruct((4, T, NP, 64), jnp.float32),
    compiler_params=pltpu.CompilerParams(use_tc_tiling_on_sc=False),
    scratch_types=[
        pltpu.VMEM((79, 2, C_CH), jnp.int32),
        pltpu.VMEM((C_CH,), jnp.int32),
        pltpu.VMEM((C_CH,), jnp.int32),
        pltpu.VMEM((C_CH,), jnp.int32),
        pltpu.VMEM((C_CH, 64), jnp.float32),
        pltpu.VMEM((C_CH, 64), jnp.float32),
        pltpu.VMEM((C_CH, 64), jnp.float32),
        pltpu.VMEM((128, 64), jnp.float32),
        pltpu.VMEM_SHARED((NP, 64), jnp.float32),
        pltpu.SemaphoreType.DMA,
        pltpu.SemaphoreType.DMA,
        pltpu.SemaphoreType.DMA,
    ],
)
def _agg_kernel(u_hbm, eidx_hbm, s_hbm,
                ebuf, adj0, adj1, adj2, rows0, rows1, rows2, zb, acc_sh,
                sem0, sem1, sem2):
    c = lax.axis_index("c")
    s = lax.axis_index("s")
    nk = jnp.where(s < 2, 79, 78)

    def pre_body(k, _):
        pltpu.sync_copy(eidx_hbm.at[s + 16 * k], ebuf.at[k])
        return 0

    lax.fori_loop(0, nk, pre_body, 0)
    _fill(zb, 128, 64, 0.0)
    for i in range(5):
        pltpu.sync_copy(zb, acc_sh.at[pl.ds(s * 640 + i * 128, 128)])
    plsc.subcore_barrier()

    def adj_of(j, off, adj):
        for i in range(C_CH // 16):
            adj[pl.ds(i * 16, 16)] = ebuf[j, 0, pl.ds(i * 16, 16)] + off

    def r_body(r, _):
        g = 2 * c + r // T
        off = g * T * N + (r % T) * N

        slots = ((adj0, rows0, sem0), (adj1, rows1, sem1), (adj2, rows2, sem2))

        for b in range(3):
            adj_of(b, off, slots[b][0])
            pltpu.async_copy(u_hbm.at[slots[b][0]], slots[b][1], slots[b][2])

        def k_body(m, _):
            for b in range(3):
                adj, rows, sem = slots[b]
                q = 3 * m + b
                pltpu.make_async_copy(u_hbm.at[adj], rows, sem).wait()
                pltpu.sync_copy(rows, acc_sh.at[ebuf.at[q, 1]], add=True)

                @pl.when(q + 3 < 78)
                def _():
                    adj_of(q + 3, off, adj)
                    pltpu.async_copy(u_hbm.at[adj], rows, sem)

            return 0

        lax.fori_loop(0, 26, k_body, 0)

        @pl.when(s < 2)
        def _():
            adj_of(78, off, adj0)
            pltpu.async_copy(u_hbm.at[adj0], rows0, sem0)
            pltpu.make_async_copy(u_hbm.at[adj0], rows0, sem0).wait()
            pltpu.sync_copy(rows0, acc_sh.at[ebuf.at[78, 1]], add=True)

        plsc.subcore_barrier()
        pltpu.sync_copy(acc_sh.at[pl.ds(s * 640, 640)],
                        s_hbm.at[g, r % T, pl.ds(s * 640, 640)])
        for i in range(5):
            pltpu.sync_copy(zb, acc_sh.at[pl.ds(s * 640 + i * 128, 128)])
        plsc.subcore_barrier()
        return 0

    lax.fori_loop(0, 2 * T, r_body, 0)


def _proj_body(xT_ref, degT_ref, Wz_ref, lzw_ref, Wh_ref, lhw_ref, out_ref):
    Wzp = jnp.dot(Wz_ref[...], lzw_ref[0:OUT, :],
                  preferred_element_type=jnp.float32)
    Whp = jnp.dot(Wh_ref[...], lhw_ref[0:OUT, :],
                  preferred_element_type=jnp.float32)
    d = degT_ref[:, 0:1] + degT_ref[:, 1:2] + 1.0
    dis = lax.rsqrt(d)
    for h in range(2):
        for bl in range(2):
            xb = xT_ref[0, 2 * h + bl]
            yz = jnp.dot(xb, Wzp, preferred_element_type=jnp.float32)
            yh = jnp.dot(xb, Whp, preferred_element_type=jnp.float32)
            out_ref[2 * h, 0, :, bl * OUT:(bl + 1) * OUT] = dis * yz
            out_ref[2 * h + 1, 0, :, bl * OUT:(bl + 1) * OUT] = dis * yh


def _head_body(S_ref, U_ref, degT_ref, attn_ref,
               bz_ref, lzw_ref, lzb_ref, bh_ref, lhw_ref, lhb_ref,
               l1w_ref, l1b_ref, l2w_ref, l2b_ref, out_ref):
    a = attn_ref[0, :]
    m = jnp.max(a)
    e = jnp.exp(a - m)
    p = e / jnp.sum(e)
    cz = jnp.dot(bz_ref[...], lzw_ref[0:OUT, :],
                 preferred_element_type=jnp.float32) + lzb_ref[...]
    ch = jnp.dot(bh_ref[...], lhw_ref[0:OUT, :],
                 preferred_element_type=jnp.float32) + lhb_ref[...]
    d = degT_ref[:, 0:1] + degT_ref[:, 1:2] + 1.0
    dis = lax.rsqrt(d)
    Hacc = [jnp.zeros((NB, OUT), jnp.float32) for _ in range(B)]
    for t in range(T):
        pt = p[t]
        for h in range(2):
            Pz = dis * (S_ref[2 * h, t] + U_ref[2 * h, t])
            Phh = dis * (S_ref[2 * h + 1, t] + U_ref[2 * h + 1, t])
            for bl in range(2):
                b = 2 * h + bl
                preZ = Pz[:, bl * OUT:(bl + 1) * OUT] + cz
                preH = Phh[:, bl * OUT:(bl + 1) * OUT] + ch
                Hn = (1.0 - jax.nn.sigmoid(preZ)) * jnp.tanh(preH)
                Hacc[b] = Hacc[b] + pt * Hn
    for b in range(B):
        h = jnp.tanh(Hacc[b])
        h2 = jnp.tanh(jnp.dot(h, l1w_ref[...],
                              preferred_element_type=jnp.float32)
                      + l1b_ref[...])
        out_ref[b] = jnp.dot(h2, l2w_ref[...],
                             preferred_element_type=jnp.float32) + l2b_ref[...]


def kernel(x, edge_index, edge_features, attn, Wz, bz, Wr, br, Wh, bh,
           lz_w, lz_b, lr_w, lr_b, lh_w, lh_b, l1_w, l1_b, l2_w, l2_b):
    eidx = jnp.stack([edge_index[0].reshape(C_NCH, C_CH),
                      edge_index[1].reshape(C_NCH, C_CH)], axis=1)
    dst_a = edge_index[1].reshape(A_NCH, A_CH)
    xT = jnp.transpose(x, (3, 0, 1, 2))

    deg2 = _deg_kernel(dst_a)
    degT = jnp.transpose(deg2)

    ubar = pl.pallas_call(
        _proj_body,
        grid=(T, GRID_N),
        in_specs=[
            pl.BlockSpec((1, B, NB, F_IN), lambda t, nb: (t, 0, nb, 0)),
            pl.BlockSpec((NB, 2), lambda t, nb: (nb, 0)),
            pl.BlockSpec((F_IN, OUT), lambda t, nb: (0, 0)),
            pl.BlockSpec((2 * OUT, OUT), lambda t, nb: (0, 0)),
            pl.BlockSpec((F_IN, OUT), lambda t, nb: (0, 0)),
            pl.BlockSpec((2 * OUT, OUT), lambda t, nb: (0, 0)),
        ],
        out_specs=pl.BlockSpec((4, 1, NB, 64), lambda t, nb: (0, t, nb, 0)),
        out_shape=jax.ShapeDtypeStruct((4, T, N, 64), jnp.float32),
    )(xT, degT, Wz, lz_w, Wh, lh_w)

    u_flat = ubar.reshape(4 * T * N, 64)
    s_pad = _agg_kernel(u_flat, eidx)

    out = pl.pallas_call(
        _head_body,
        grid=(GRID_N,),
        in_specs=[
            pl.BlockSpec((4, T, NB, 64), lambda nb: (0, 0, nb, 0)),
            pl.BlockSpec((4, T, NB, 64), lambda nb: (0, 0, nb, 0)),
            pl.BlockSpec((NB, 2), lambda nb: (nb, 0)),
            pl.BlockSpec((1, T), lambda nb: (0, 0)),
            pl.BlockSpec((1, OUT), lambda nb: (0, 0)),
            pl.BlockSpec((2 * OUT, OUT), lambda nb: (0, 0)),
            pl.BlockSpec((1, OUT), lambda nb: (0, 0)),
            pl.BlockSpec((1, OUT), lambda nb: (0, 0)),
            pl.BlockSpec((2 * OUT, OUT), lambda nb: (0, 0)),
            pl.BlockSpec((1, OUT), lambda nb: (0, 0)),
            pl.BlockSpec((OUT, 16), lambda nb: (0, 0)),
            pl.BlockSpec((1, 16), lambda nb: (0, 0)),
            pl.BlockSpec((16, 12), lambda nb: (0, 0)),
            pl.BlockSpec((1, 12), lambda nb: (0, 0)),
        ],
        out_specs=pl.BlockSpec((B, NB, 12), lambda nb: (0, nb, 0)),
        out_shape=jax.ShapeDtypeStruct((B, N, 12), jnp.float32),
    )(s_pad, ubar, degT, attn.reshape(1, T),
      bz.reshape(1, OUT), lz_w, lz_b.reshape(1, OUT),
      bh.reshape(1, OUT), lh_w, lh_b.reshape(1, OUT),
      l1_w, l1_b.reshape(1, 16), l2_w, l2_b.reshape(1, 12))
    return out


# speedup vs baseline: 311.1146x; 1.1625x over previous
import functools

import jax
import jax.numpy as jnp
from jax import lax
from jax.experimental import pallas as pl
from jax.experimental.pallas import tpu as pltpu
from jax.experimental.pallas import tpu_sc as plsc

N = 10000
NP = 10240
E = 160000
B = 4
F_IN = 128
T = 12
OUT = 32
NB = 200
GRID_N = N // NB

C_CH = 64
C_NCH = 2500
A_CH = 128
A_NCH = E // A_CH

_MESH = plsc.VectorSubcoreMesh(core_axis_name="c", subcore_axis_name="s")


def _fill1d(ref, n, val, dtype=jnp.float32):
    def body(r, _):
        ref[pl.ds(r * 16, 16)] = jnp.full((16,), val, dtype)
        return 0
    lax.fori_loop(0, n // 16, body, 0)


@functools.partial(
    pl.kernel,
    mesh=_MESH,
    out_type=jax.ShapeDtypeStruct((2, NP), jnp.float32),
    scratch_types=[
        pltpu.VMEM((A_CH,), jnp.int32),
        pltpu.VMEM((A_CH,), jnp.float32),
        pltpu.VMEM((640,), jnp.float32),
        pltpu.VMEM_SHARED((NP,), jnp.float32),
    ],
)
def _deg_kernel(dst_hbm, deg_hbm, idx_v, ones_v, zb_v, acc_sh):
    c = lax.axis_index("c")
    s = lax.axis_index("s")
    w = c * 16 + s
    _fill1d(ones_v, A_CH, 1.0)
    _fill1d(zb_v, 640, 0.0)
    pltpu.sync_copy(zb_v, acc_sh.at[pl.ds(s * 640, 640)])
    plsc.subcore_barrier()
    nch = jnp.where(w < 2, 40, 39)

    def body(k, _):
        ci = w + 32 * k
        pltpu.sync_copy(dst_hbm.at[ci], idx_v)
        pltpu.sync_copy(ones_v, acc_sh.at[idx_v], add=True)
        return 0

    lax.fori_loop(0, nch, body, 0)
    plsc.subcore_barrier()
    pltpu.sync_copy(acc_sh.at[pl.ds(s * 640, 640)],
                    deg_hbm.at[c, pl.ds(s * 640, 640)])


@functools.partial(
    pl.kernel,
    mesh=_MESH,
    out_type=jax.ShapeDtypeStruct((2, T, NP, 128), jnp.float32),
    scratch_types=[
        pltpu.VMEM((2, C_CH), jnp.int32),
        pltpu.VMEM((2, C_CH), jnp.int32),
        pltpu.VMEM((2, C_CH), jnp.int32),
        pltpu.VMEM((C_CH,), jnp.int32),
        pltpu.VMEM((C_CH,), jnp.int32),
        pltpu.VMEM((C_CH,), jnp.int32),
        pltpu.VMEM((C_CH,), jnp.int32),
        pltpu.VMEM((C_CH,), jnp.int32),
        pltpu.VMEM((C_CH,), jnp.int32),
        pltpu.VMEM((C_CH, 128), jnp.float32),
        pltpu.VMEM((C_CH, 128), jnp.float32),
        pltpu.VMEM((C_CH, 128), jnp.float32),
        pltpu.VMEM_SHARED((NP, 128), jnp.float32),
        pltpu.SemaphoreType.DMA,
        pltpu.SemaphoreType.DMA,
        pltpu.SemaphoreType.DMA,
        pltpu.SemaphoreType.DMA,
        pltpu.SemaphoreType.DMA,
        pltpu.SemaphoreType.DMA,
    ],
)
def _agg_kernel(u_hbm, eidx_hbm, zrow_hbm, s_hbm,
                eb0, eb1, eb2, adj0, adj1, adj2, dsti0, dsti1, dsti2,
                rows0, rows1, rows2, acc_sh,
                sg0, sg1, sg2, si0, si1, si2):
    c = lax.axis_index("c")
    s = lax.axis_index("s")
    slots = ((eb0, adj0, dsti0, rows0, sg0, si0),
             (eb1, adj1, dsti1, rows1, sg1, si1),
             (eb2, adj2, dsti2, rows2, sg2, si2))

    def zero_acc():
        for i in range(5):
            pltpu.sync_copy(zrow_hbm, acc_sh.at[pl.ds(s * 640 + i * 128, 128)])

    zero_acc()
    plsc.subcore_barrier()

    def extract(eb, adj, dsti, off):
        for i in range(C_CH // 16):
            adj[pl.ds(i * 16, 16)] = eb[0, pl.ds(i * 16, 16)] + off
            dsti[pl.ds(i * 16, 16)] = eb[1, pl.ds(i * 16, 16)]

    def has_chunk(k):
        return jnp.logical_or(k < 156, jnp.logical_and(k == 156, s < 4))

    def t_body(t, _):
        off = (c * T + t) * N

        for b in range(3):
            eb, adj, dsti, rows, sg, si = slots[b]
            pltpu.sync_copy(eidx_hbm.at[s + 16 * b], eb)
            extract(eb, adj, dsti, off)
            pltpu.async_copy(u_hbm.at[adj], rows, sg)
            pltpu.async_copy(eidx_hbm.at[s + 16 * (b + 3)], eb, si)

        def m_body(m, _):
            for b in range(3):
                eb, adj, dsti, rows, sg, si = slots[b]
                q = 3 * m + b
                pltpu.make_async_copy(u_hbm.at[adj], rows, sg).wait()
                pltpu.sync_copy(rows, acc_sh.at[dsti], add=True)

                @pl.when(has_chunk(q + 3))
                def _():
                    pltpu.make_async_copy(eidx_hbm.at[s], eb, si).wait()
                    extract(eb, adj, dsti, off)
                    pltpu.async_copy(u_hbm.at[adj], rows, sg)

                @pl.when(has_chunk(q + 6))
                def _():
                    pltpu.async_copy(eidx_hbm.at[s + 16 * (q + 6)], eb, si)

            return 0

        lax.fori_loop(0, 52, m_body, 0)

        @pl.when(s < 4)
        def _():
            pltpu.make_async_copy(u_hbm.at[adj0], rows0, sg0).wait()
            pltpu.sync_copy(rows0, acc_sh.at[dsti0], add=True)

        plsc.subcore_barrier()
        pltpu.sync_copy(acc_sh.at[pl.ds(s * 640, 640)],
                        s_hbm.at[c, t, pl.ds(s * 640, 640)])
        zero_acc()
        plsc.subcore_barrier()
        return 0

    lax.fori_loop(0, T, t_body, 0)


def _proj_body(xT_ref, degT_ref, Wz_ref, lzw_ref, Wh_ref, lhw_ref, out_ref):
    Wzp = jnp.dot(Wz_ref[...], lzw_ref[0:OUT, :],
                  preferred_element_type=jnp.float32)
    Whp = jnp.dot(Wh_ref[...], lhw_ref[0:OUT, :],
                  preferred_element_type=jnp.float32)
    d = degT_ref[:, 0:1] + degT_ref[:, 1:2] + 1.0
    dis = lax.rsqrt(d)
    for h in range(2):
        for bl in range(2):
            xb = xT_ref[0, 2 * h + bl]
            yz = jnp.dot(xb, Wzp, preferred_element_type=jnp.float32)
            yh = jnp.dot(xb, Whp, preferred_element_type=jnp.float32)
            out_ref[h, 0, :, bl * 64:bl * 64 + OUT] = dis * yz
            out_ref[h, 0, :, bl * 64 + OUT:bl * 64 + 64] = dis * yh


def _head_body(S_ref, U_ref, degT_ref, attn_ref,
               bz_ref, lzw_ref, lzb_ref, bh_ref, lhw_ref, lhb_ref,
               l1w_ref, l1b_ref, l2w_ref, l2b_ref, out_ref):
    a = attn_ref[0, :]
    m = jnp.max(a)
    e = jnp.exp(a - m)
    p = e / jnp.sum(e)
    cz = jnp.dot(bz_ref[...], lzw_ref[0:OUT, :],
                 preferred_element_type=jnp.float32) + lzb_ref[...]
    ch = jnp.dot(bh_ref[...], lhw_ref[0:OUT, :],
                 preferred_element_type=jnp.float32) + lhb_ref[...]
    d = degT_ref[:, 0:1] + degT_ref[:, 1:2] + 1.0
    dis = lax.rsqrt(d)
    Hacc = [jnp.zeros((NB, OUT), jnp.float32) for _ in range(B)]
    for t in range(T):
        pt = p[t]
        for h in range(2):
            Prow = dis * (S_ref[h, t] + U_ref[h, t])
            for bl in range(2):
                b = 2 * h + bl
                preZ = Prow[:, bl * 64:bl * 64 + OUT] + cz
                preH = Prow[:, bl * 64 + OUT:bl * 64 + 64] + ch
                Hn = (1.0 - jax.nn.sigmoid(preZ)) * jnp.tanh(preH)
                Hacc[b] = Hacc[b] + pt * Hn
    for b in range(B):
        h = jnp.tanh(Hacc[b])
        h2 = jnp.tanh(jnp.dot(h, l1w_ref[...],
                              preferred_element_type=jnp.float32)
                      + l1b_ref[...])
        out_ref[b] = jnp.dot(h2, l2w_ref[...],
                             preferred_element_type=jnp.float32) + l2b_ref[...]


def kernel(x, edge_index, edge_features, attn, Wz, bz, Wr, br, Wh, bh,
           lz_w, lz_b, lr_w, lr_b, lh_w, lh_b, l1_w, l1_b, l2_w, l2_b):
    eidx = jnp.stack([edge_index[0].reshape(C_NCH, C_CH),
                      edge_index[1].reshape(C_NCH, C_CH)], axis=1)
    dst_a = edge_index[1].reshape(A_NCH, A_CH)
    zrow = jnp.zeros((128, 128), jnp.float32)
    xT = jnp.transpose(x, (3, 0, 1, 2))

    deg2 = _deg_kernel(dst_a)
    degT = jnp.transpose(deg2)

    ubar = pl.pallas_call(
        _proj_body,
        grid=(T, GRID_N),
        in_specs=[
            pl.BlockSpec((1, B, NB, F_IN), lambda t, nb: (t, 0, nb, 0)),
            pl.BlockSpec((NB, 2), lambda t, nb: (nb, 0)),
            pl.BlockSpec((F_IN, OUT), lambda t, nb: (0, 0)),
            pl.BlockSpec((2 * OUT, OUT), lambda t, nb: (0, 0)),
            pl.BlockSpec((F_IN, OUT), lambda t, nb: (0, 0)),
            pl.BlockSpec((2 * OUT, OUT), lambda t, nb: (0, 0)),
        ],
        out_specs=pl.BlockSpec((2, 1, NB, 128), lambda t, nb: (0, t, nb, 0)),
        out_shape=jax.ShapeDtypeStruct((2, T, N, 128), jnp.float32),
    )(xT, degT, Wz, lz_w, Wh, lh_w)

    u_flat = ubar.reshape(2 * T * N, 128)
    s_out = _agg_kernel(u_flat, eidx, zrow)

    out = pl.pallas_call(
        _head_body,
        grid=(GRID_N,),
        in_specs=[
            pl.BlockSpec((2, T, NB, 128), lambda nb: (0, 0, nb, 0)),
            pl.BlockSpec((2, T, NB, 128), lambda nb: (0, 0, nb, 0)),
            pl.BlockSpec((NB, 2), lambda nb: (nb, 0)),
            pl.BlockSpec((1, T), lambda nb: (0, 0)),
            pl.BlockSpec((1, OUT), lambda nb: (0, 0)),
            pl.BlockSpec((2 * OUT, OUT), lambda nb: (0, 0)),
            pl.BlockSpec((1, OUT), lambda nb: (0, 0)),
            pl.BlockSpec((1, OUT), lambda nb: (0, 0)),
            pl.BlockSpec((2 * OUT, OUT), lambda nb: (0, 0)),
            pl.BlockSpec((1, OUT), lambda nb: (0, 0)),
            pl.BlockSpec((OUT, 16), lambda nb: (0, 0)),
            pl.BlockSpec((1, 16), lambda nb: (0, 0)),
            pl.BlockSpec((16, 12), lambda nb: (0, 0)),
            pl.BlockSpec((1, 12), lambda nb: (0, 0)),
        ],
        out_specs=pl.BlockSpec((B, NB, 12), lambda nb: (0, nb, 0)),
        out_shape=jax.ShapeDtypeStruct((B, N, 12), jnp.float32),
    )(s_out, ubar, degT, attn.reshape(1, T),
      bz.reshape(1, OUT), lz_w, lz_b.reshape(1, OUT),
      bh.reshape(1, OUT), lh_w, lh_b.reshape(1, OUT),
      l1_w, l1_b.reshape(1, 16), l2_w, l2_b.reshape(1, 12))
    return out
